# Initial kernel scaffold; baseline (speedup 1.0000x reference)
import jax
import jax.numpy as jnp
from jax.experimental import pallas as pl


def kernel(edge_index, x, W_self0, W_neigh0, b0, W_self1, W_neigh1, b1, W_self2, W_neigh2, b2):
    raise NotImplementedError("write your pallas kernel here")


# speedup vs baseline: 3.0968x; 3.0968x over previous
import functools

import jax
import jax.numpy as jnp
from jax import lax
from jax.experimental import pallas as pl
from jax.experimental.pallas import tpu as pltpu
from jax.experimental.pallas import tpu_sc as plsc

N = 10000
E = 320000
D = 128

NC = 2
NS = 16
NW = NC * NS

CHUNK = 128
NCHUNK = 80
EPW = NCHUNK * CHUNK
E_PAD = EPW * NW
N_PAD = N + 112
RPS = N_PAD // NS
DROWS = 80

BM = 2000


def _sc_stage_and_zero(src_hbm, dst_hbm, z_hbm, src_v, dst_v, acc, sid, wid):
    pltpu.sync_copy(src_hbm.at[wid], src_v)
    pltpu.sync_copy(dst_hbm.at[wid], dst_v)
    pltpu.sync_copy(z_hbm.at[pl.ds(sid * RPS, RPS)],
                    acc.at[pl.ds(sid * RPS, RPS)])


def _sc_main_loop(y_hbm, src_v, dst_v, buf0, acc):
    def body(j, carry):
        pltpu.sync_copy(y_hbm.at[src_v.at[j]], buf0)
        pltpu.sync_copy(buf0, acc.at[dst_v.at[j]], add=True)
        return carry

    lax.fori_loop(0, NCHUNK, body, 0)


def _sc_agg_body(y_hbm, src_hbm, dst_hbm, z_hbm, out_hbm, src_v, dst_v,
                 buf0, acc):
    cid = lax.axis_index("c")
    sid = lax.axis_index("s")
    wid = sid * NC + cid
    _sc_stage_and_zero(src_hbm, dst_hbm, z_hbm, src_v, dst_v, acc, sid, wid)
    plsc.subcore_barrier()
    _sc_main_loop(y_hbm, src_v, dst_v, buf0, acc)
    plsc.subcore_barrier()
    pltpu.sync_copy(acc.at[pl.ds(sid * RPS, RPS)],
                    out_hbm.at[cid, pl.ds(sid * RPS, RPS)])


def _sc_agg_deg_body(y_hbm, src_hbm, dst_hbm, z_hbm, ident_hbm,
                     out_hbm, deg_hbm,
                     src_v, dst_v, buf0, degbuf, ident_v, acc, accd):
    cid = lax.axis_index("c")
    sid = lax.axis_index("s")
    wid = sid * NC + cid
    _sc_stage_and_zero(src_hbm, dst_hbm, z_hbm, src_v, dst_v, acc, sid, wid)
    pltpu.sync_copy(ident_hbm, ident_v)

    @pl.when(sid < DROWS // 8)
    def _():
        pltpu.sync_copy(z_hbm.at[pl.ds(sid * 8, 8)],
                        accd.at[pl.ds(sid * 8, 8)])

    def zb(k, carry):
        degbuf[k >> 3, pl.ds((k & 7) * 16, 16)] = jnp.zeros((16,), jnp.float32)
        return carry

    lax.fori_loop(0, DROWS * 8, zb, 0)

    iota = lax.iota(jnp.int32, 16)
    ones_f = jnp.ones((16,), jnp.float32)
    lane_masks = [iota == k for k in range(16)]

    def hist(e, carry):
        d = dst_v[e >> 3, pl.ds((e & 7) * 16, 16)]
        r = d >> 7
        c = d & 127
        for k in range(16):
            plsc.addupdate_scatter(degbuf, [r, c], ones_f, mask=lane_masks[k])
        return carry

    lax.fori_loop(0, EPW // 16, hist, 0)
    plsc.subcore_barrier()
    _sc_main_loop(y_hbm, src_v, dst_v, buf0, acc)
    pltpu.sync_copy(degbuf, accd.at[ident_v.at[0]], add=True)
    plsc.subcore_barrier()
    pltpu.sync_copy(acc.at[pl.ds(sid * RPS, RPS)],
                    out_hbm.at[cid, pl.ds(sid * RPS, RPS)])

    @pl.when(sid < DROWS // 8)
    def _():
        pltpu.sync_copy(accd.at[pl.ds(sid * 8, 8)],
                        deg_hbm.at[cid, pl.ds(sid * 8, 8)])


_MESH = plsc.VectorSubcoreMesh(core_axis_name="c", subcore_axis_name="s")


@functools.lru_cache(maxsize=None)
def _make_sc_agg():
    return pl.kernel(
        _sc_agg_body,
        mesh=_MESH,
        compiler_params=pltpu.CompilerParams(needs_layout_passes=False),
        out_type=jax.ShapeDtypeStruct((NC, N_PAD, D), jnp.float32),
        scratch_types=[
            pltpu.VMEM((NCHUNK, CHUNK), jnp.int32),
            pltpu.VMEM((NCHUNK, CHUNK), jnp.int32),
            pltpu.VMEM((CHUNK, D), jnp.float32),
            pltpu.VMEM_SHARED((N_PAD, D), jnp.float32),
        ],
    )


@functools.lru_cache(maxsize=None)
def _make_sc_agg_deg():
    return pl.kernel(
        _sc_agg_deg_body,
        mesh=_MESH,
        compiler_params=pltpu.CompilerParams(needs_layout_passes=False),
        out_type=(
            jax.ShapeDtypeStruct((NC, N_PAD, D), jnp.float32),
            jax.ShapeDtypeStruct((NC, DROWS, 128), jnp.float32),
        ),
        scratch_types=[
            pltpu.VMEM((NCHUNK, CHUNK), jnp.int32),
            pltpu.VMEM((NCHUNK, CHUNK), jnp.int32),
            pltpu.VMEM((CHUNK, D), jnp.float32),
            pltpu.VMEM((DROWS, 128), jnp.float32),
            pltpu.VMEM((1, DROWS), jnp.int32),
            pltpu.VMEM_SHARED((N_PAD, D), jnp.float32),
            pltpu.VMEM_SHARED((DROWS, 128), jnp.float32),
        ],
    )


def _mm_body(h_ref, wn_ref, ws_ref, b_ref, y_ref, s_ref):
    h = h_ref[...]
    y_ref[...] = jnp.dot(h, wn_ref[...], preferred_element_type=jnp.float32)
    s_ref[...] = jnp.dot(h, ws_ref[...],
                         preferred_element_type=jnp.float32) + b_ref[...]


def _mm(h, wn, ws, b):
    n, k = h.shape
    wy = wn.shape[1]
    wsc = ws.shape[1]
    grid = (n // BM,)
    return pl.pallas_call(
        _mm_body,
        grid=grid,
        in_specs=[
            pl.BlockSpec((BM, k), lambda i: (i, 0)),
            pl.BlockSpec(wn.shape, lambda i: (0, 0)),
            pl.BlockSpec(ws.shape, lambda i: (0, 0)),
            pl.BlockSpec((1, wsc), lambda i: (0, 0)),
        ],
        out_specs=[
            pl.BlockSpec((BM, wy), lambda i: (i, 0)),
            pl.BlockSpec((BM, wsc), lambda i: (i, 0)),
        ],
        out_shape=[
            jax.ShapeDtypeStruct((n, wy), jnp.float32),
            jax.ShapeDtypeStruct((n, wsc), jnp.float32),
        ],
    )(h, wn, ws, b.reshape(1, -1))


def _combine0_body(s_ref, p_ref, d_ref, h_ref, r_ref):
    p = p_ref[0] + p_ref[1]
    deg = d_ref[0] + d_ref[1]
    recip = 1.0 / jnp.maximum(deg, 1.0)
    h_ref[...] = jnp.maximum(s_ref[...] + p * recip, 0.0)
    r_ref[...] = jnp.broadcast_to(recip, (s_ref.shape[0], 16))


def _combine0(s, partials, pdeg):
    grid = (N // BM,)
    return pl.pallas_call(
        _combine0_body,
        grid=grid,
        in_specs=[
            pl.BlockSpec((BM, D), lambda i: (i, 0)),
            pl.BlockSpec((NC, BM, D), lambda i: (0, i, 0)),
            pl.BlockSpec((NC, BM, 1), lambda i: (0, i, 0)),
        ],
        out_specs=[
            pl.BlockSpec((BM, D), lambda i: (i, 0)),
            pl.BlockSpec((BM, 16), lambda i: (i, 0)),
        ],
        out_shape=[
            jax.ShapeDtypeStruct((N, D), jnp.float32),
            jax.ShapeDtypeStruct((N, 16), jnp.float32),
        ],
    )(s, partials, pdeg)


def _combine_body_relu(s_ref, p_ref, r_ref, o_ref):
    p = p_ref[0] + p_ref[1]
    o_ref[...] = jnp.maximum(s_ref[...] + p * r_ref[:, :1], 0.0)


def _combine_body_lin(s_ref, p_ref, r_ref, o_ref):
    p = p_ref[0] + p_ref[1]
    o_ref[...] = s_ref[...] + p * r_ref[:, :1]


def _combine(s, partials, recip, relu):
    w = s.shape[1]
    grid = (N // BM,)
    return pl.pallas_call(
        _combine_body_relu if relu else _combine_body_lin,
        grid=grid,
        in_specs=[
            pl.BlockSpec((BM, w), lambda i: (i, 0)),
            pl.BlockSpec((NC, BM, w), lambda i: (0, i, 0)),
            pl.BlockSpec((BM, 16), lambda i: (i, 0)),
        ],
        out_specs=pl.BlockSpec((BM, w), lambda i: (i, 0)),
        out_shape=jax.ShapeDtypeStruct((N, w), jnp.float32),
    )(s, partials, recip)


def kernel(edge_index, x, W_self0, W_neigh0, b0, W_self1, W_neigh1, b1,
           W_self2, W_neigh2, b2):
    ei = edge_index.astype(jnp.int32)
    pad = E_PAD - E
    src_p = jnp.concatenate([ei[0], jnp.zeros((pad,), jnp.int32)])
    dst_p = jnp.concatenate(
        [ei[1], N + (jnp.arange(pad, dtype=jnp.int32) % 16)])
    src3 = src_p.reshape(NW, NCHUNK, CHUNK)
    dst3 = dst_p.reshape(NW, NCHUNK, CHUNK)

    zeros = jnp.zeros((N_PAD, D), jnp.float32)
    ident = jnp.arange(DROWS, dtype=jnp.int32).reshape(1, DROWS)

    wn2 = jnp.pad(W_neigh2, ((0, 0), (0, D - W_neigh2.shape[1])))
    ws2 = jnp.pad(W_self2, ((0, 0), (0, D - W_self2.shape[1])))
    b2p = jnp.pad(b2, (0, D - b2.shape[0]))

    sc_agg = _make_sc_agg()

    y0, s0 = _mm(x, W_neigh0, W_self0, b0)
    p0, pdeg = _make_sc_agg_deg()(y0, src3, dst3, zeros, ident)
    h1, recip = _combine0(s0, p0, pdeg.reshape(NC, DROWS * 128, 1))
    y1, s1 = _mm(h1, W_neigh1, W_self1, b1)
    p1 = sc_agg(y1, src3, dst3, zeros)
    h2 = _combine(s1, p1, recip, relu=True)
    y2, s2 = _mm(h2, wn2, ws2, b2p)
    p2 = sc_agg(y2, src3, dst3, zeros)
    out = _combine(s2, p2, recip, relu=False)
    return out[:, :47]
